# baseline (speedup 1.0000x reference)
import functools

import jax
import jax.numpy as jnp
from jax import lax
from jax.experimental import pallas as pl
from jax.experimental.pallas import tpu as pltpu
from jax.experimental.pallas import tpu_sc as plsc

B = 16384
L = 20
D = 32
PK = 128

NC, NS = 2, 16
NW = NC * NS
SPW = B // NW

CH = 16
NCH = SPW // CH
CHB = 64
NCHB = SPW // CHB

_MESH = plsc.VectorSubcoreMesh(
    core_axis_name="c", subcore_axis_name="s", num_cores=NC, num_subcores=NS)


@functools.partial(
    pl.kernel,
    out_type=jax.ShapeDtypeStruct((B * D,), jnp.float32),
    mesh=_MESH,
    compiler_params=pltpu.CompilerParams(needs_layout_passes=False),
    scratch_types=[
        pltpu.VMEM((L * CH,), jnp.int32),
        pltpu.VMEM((L * CH,), jnp.int32),
        pltpu.VMEM((L * CH, PK), jnp.float32),
        pltpu.VMEM((L * CH, PK), jnp.float32),
        pltpu.VMEM((SPW * D,), jnp.float32),
        pltpu.SemaphoreType.DMA,
        pltpu.SemaphoreType.DMA,
    ],
)
def _text_kernel(text_hbm, tb_hbm, out_hbm, ids0, ids1, rows0, rows1, outw,
                 isem, gsem):
    wid = lax.axis_index("s") * NC + lax.axis_index("c")
    cid0 = wid * NCH

    def stage_idx(cidx, ib):
        pltpu.async_copy(tb_hbm.at[pl.ds(cidx * (L * CH), L * CH)], ib, isem)

    def wait_idx(cidx, ib):
        pltpu.make_async_copy(
            tb_hbm.at[pl.ds(cidx * (L * CH), L * CH)], ib, isem).wait()

    def fire_gathers(ib, rb):
        for j in range(L):
            pltpu.async_copy(text_hbm.at[ib.at[pl.ds(j * CH, CH)]],
                             rb.at[pl.ds(j * CH, CH)], gsem)

    def wait_gathers(ib, rb):
        for j in range(L):
            pltpu.make_async_copy(text_hbm.at[ib.at[pl.ds(j * CH, CH)]],
                                  rb.at[pl.ds(j * CH, CH)], gsem).wait()

    pltpu.sync_copy(tb_hbm.at[pl.ds(cid0 * (L * CH), L * CH)], ids0)
    fire_gathers(ids0, rows0)
    stage_idx(cid0 + 1, ids1)

    bufs = ((ids0, rows0), (ids1, rows1))

    def body(cc, _):
        for p in range(2):
            c = cc * 2 + p
            cidx = cid0 + c
            ib, rb = bufs[p]
            ibn, rbn = bufs[1 - p]

            @pl.when(c + 1 < NCH)
            def _():
                wait_idx(cidx + 1, ibn)
                fire_gathers(ibn, rbn)

            iot = lax.iota(jnp.int32, 16) * L
            cnt = jnp.zeros((16,), jnp.float32)
            for j in range(L):
                iv = plsc.load_gather(ib, [iot + j])
                cnt = cnt + jnp.where(iv != 0, 1.0, 0.0)
            rvec = 1.0 / jnp.maximum(cnt, 1.0)

            @pl.when(c + 2 < NCH)
            def _():
                stage_idx(cidx + 2, ib)

            wait_gathers(ib, rb)

            for i2 in range(16):
                a0 = rb[i2 * L, pl.ds(0, 16)]
                a1 = rb[i2 * L, pl.ds(16, 16)]
                for j in range(1, L):
                    a0 = a0 + rb[i2 * L + j, pl.ds(0, 16)]
                    a1 = a1 + rb[i2 * L + j, pl.ds(16, 16)]
                r = rvec[i2]
                ob = pl.multiple_of(c * (CH * D) + i2 * D, D)
                outw[pl.ds(ob, 16)] = a0 * r
                outw[pl.ds(ob + 16, 16)] = a1 * r
        return 0

    lax.fori_loop(0, NCH // 2, body, 0)
    pltpu.sync_copy(outw, out_hbm.at[pl.ds(wid * (SPW * D), SPW * D)])


@functools.partial(
    pl.kernel,
    out_type=jax.ShapeDtypeStruct((B * 2 * D,), jnp.float32),
    mesh=_MESH,
    compiler_params=pltpu.CompilerParams(needs_layout_passes=False),
    scratch_types=[
        pltpu.VMEM((CHB,), jnp.int32),
        pltpu.VMEM((CHB,), jnp.int32),
        pltpu.VMEM((CHB, PK), jnp.float32),
        pltpu.VMEM((CHB, PK), jnp.float32),
        pltpu.VMEM((CHB * D,), jnp.float32),
        pltpu.VMEM((CHB * D,), jnp.float32),
        pltpu.VMEM((SPW * 2 * D,), jnp.float32),
        pltpu.SemaphoreType.DMA,
        pltpu.SemaphoreType.DMA,
    ],
)
def _title_kernel(title_hbm, tids_hbm, text_hbm, out_hbm,
                  tix0, tix1, trow0, trow1, tex0, tex1, outw, isem, tsem):
    wid = lax.axis_index("s") * NC + lax.axis_index("c")
    cid0 = wid * NCHB

    def stage_idx(cidx, xb, eb):
        pltpu.async_copy(tids_hbm.at[pl.ds(cidx * CHB, CHB)], xb, isem)
        pltpu.async_copy(
            text_hbm.at[pl.ds(cidx * (CHB * D), CHB * D)], eb, isem)

    def wait_idx(cidx, xb, eb):
        pltpu.make_async_copy(
            tids_hbm.at[pl.ds(cidx * CHB, CHB)], xb, isem).wait()
        pltpu.make_async_copy(
            text_hbm.at[pl.ds(cidx * (CHB * D), CHB * D)], eb, isem).wait()

    def fire_gather(xb, tb):
        pltpu.async_copy(title_hbm.at[xb], tb, tsem)

    def wait_gather(xb, tb):
        pltpu.make_async_copy(title_hbm.at[xb], tb, tsem).wait()

    pltpu.sync_copy(tids_hbm.at[pl.ds(cid0 * CHB, CHB)], tix0)
    pltpu.sync_copy(text_hbm.at[pl.ds(cid0 * (CHB * D), CHB * D)], tex0)
    fire_gather(tix0, trow0)
    stage_idx(cid0 + 1, tix1, tex1)

    bufs = ((tix0, trow0, tex0), (tix1, trow1, tex1))

    def body(cc, _):
        for p in range(2):
            c = cc * 2 + p
            cidx = cid0 + c
            xb, tb, eb = bufs[p]
            xbn, tbn, ebn = bufs[1 - p]

            @pl.when(c + 1 < NCHB)
            def _():
                wait_idx(cidx + 1, xbn, ebn)
                fire_gather(xbn, tbn)

            wait_gather(xb, tb)

            for i2 in range(CHB):
                ob = pl.multiple_of(
                    c * (CHB * 2 * D) + i2 * 2 * D, 2 * D)
                outw[pl.ds(ob, 16)] = tb[i2, pl.ds(0, 16)]
                outw[pl.ds(ob + 16, 16)] = tb[i2, pl.ds(16, 16)]
                outw[pl.ds(ob + 32, 16)] = eb[pl.ds(i2 * D, 16)]
                outw[pl.ds(ob + 48, 16)] = eb[pl.ds(i2 * D + 16, 16)]

            @pl.when(c + 2 < NCHB)
            def _():
                stage_idx(cidx + 2, xb, eb)
        return 0

    lax.fori_loop(0, NCHB // 2, body, 0)
    pltpu.sync_copy(outw, out_hbm.at[pl.ds(wid * (SPW * 2 * D), SPW * 2 * D)])


def kernel(title_table, text_table, title_ids, token_ids):
    text_z = text_table.at[0].set(0.0)
    text_p = jnp.pad(text_z, ((0, 0), (0, PK - D)))
    title_p = jnp.pad(title_table, ((0, 7), (0, PK - D)))
    tb = token_ids.reshape(-1)
    text_flat = _text_kernel(text_p, tb)
    flat = _title_kernel(title_p, title_ids, text_flat)
    return flat.reshape(B, 2 * D)


# speedup vs baseline: 1.2558x; 1.2558x over previous
import functools

import jax
import jax.numpy as jnp
from jax import lax
from jax.experimental import pallas as pl
from jax.experimental.pallas import tpu as pltpu
from jax.experimental.pallas import tpu_sc as plsc

B = 16384
L = 20
D = 32
PK = 128

NC, NS = 2, 16
NW = NC * NS
SPW = B // NW

CH = 16
NCH = SPW // CH
CHB = 64
NCHB = SPW // CHB

_MESH = plsc.VectorSubcoreMesh(
    core_axis_name="c", subcore_axis_name="s", num_cores=NC, num_subcores=NS)


@functools.partial(
    pl.kernel,
    out_type=jax.ShapeDtypeStruct((B * D,), jnp.float32),
    mesh=_MESH,
    compiler_params=pltpu.CompilerParams(needs_layout_passes=False),
    scratch_types=[
        pltpu.VMEM_SHARED((10000, D), jnp.float32),
        pltpu.VMEM((CH, L), jnp.int32),
        pltpu.VMEM((CH, L), jnp.int32),
        pltpu.VMEM((L * CH, D), jnp.float32),
        pltpu.VMEM((L * CH, D), jnp.float32),
        pltpu.VMEM((SPW * D,), jnp.float32),
        pltpu.SemaphoreType.DMA,
        pltpu.SemaphoreType.DMA,
        pltpu.SemaphoreType.DMA,
    ],
)
def _text_kernel(text_hbm, tids2_hbm, out_hbm, tabS, ids0, ids1,
                 rows0, rows1, outw, isem, gsem, lsem):
    wid = lax.axis_index("s") * NC + lax.axis_index("c")
    cid0 = wid * NCH

    @pl.when(lax.axis_index("s") == 0)
    def _():
        pltpu.async_copy(text_hbm, tabS, lsem).wait()
    plsc.subcore_barrier()

    def stage_idx(cidx, ib):
        pltpu.async_copy(
            tids2_hbm.at[pl.ds(cidx * CH, CH), :], ib, isem)

    def wait_idx(cidx, ib):
        pltpu.make_async_copy(
            tids2_hbm.at[pl.ds(cidx * CH, CH), :], ib, isem).wait()

    def fire_gathers(ib, rb):
        for i in range(CH):
            pltpu.async_copy(tabS.at[ib.at[i]],
                             rb.at[pl.ds(i * L, L)], gsem)

    def wait_gathers(ib, rb):
        for i in range(CH):
            pltpu.make_async_copy(tabS.at[ib.at[i]],
                                  rb.at[pl.ds(i * L, L)], gsem).wait()

    pltpu.sync_copy(tids2_hbm.at[pl.ds(cid0 * CH, CH), :], ids0)
    fire_gathers(ids0, rows0)
    stage_idx(cid0 + 1, ids1)

    bufs = ((ids0, rows0), (ids1, rows1))

    def body(cc, _):
        for p in range(2):
            c = cc * 2 + p
            cidx = cid0 + c
            ib, rb = bufs[p]
            ibn, rbn = bufs[1 - p]

            @pl.when(c + 1 < NCH)
            def _():
                wait_idx(cidx + 1, ibn)
                fire_gathers(ibn, rbn)

            iot = lax.iota(jnp.int32, 16)
            cnt = jnp.zeros((16,), jnp.float32)
            for j in range(L):
                iv = plsc.load_gather(
                    ib, [iot, jnp.full((16,), j, jnp.int32)])
                cnt = cnt + jnp.where(iv != 0, 1.0, 0.0)
            rvec = 1.0 / jnp.maximum(cnt, 1.0)

            @pl.when(c + 2 < NCH)
            def _():
                stage_idx(cidx + 2, ib)

            wait_gathers(ib, rb)

            for i2 in range(16):
                a0 = rb[i2 * L, pl.ds(0, 16)]
                a1 = rb[i2 * L, pl.ds(16, 16)]
                for j in range(1, L):
                    a0 = a0 + rb[i2 * L + j, pl.ds(0, 16)]
                    a1 = a1 + rb[i2 * L + j, pl.ds(16, 16)]
                r = rvec[i2]
                ob = pl.multiple_of(c * (CH * D) + i2 * D, D)
                outw[pl.ds(ob, 16)] = a0 * r
                outw[pl.ds(ob + 16, 16)] = a1 * r
        return 0

    lax.fori_loop(0, NCH // 2, body, 0)
    pltpu.sync_copy(outw, out_hbm.at[pl.ds(wid * (SPW * D), SPW * D)])


@functools.partial(
    pl.kernel,
    out_type=jax.ShapeDtypeStruct((B * 2 * D,), jnp.float32),
    mesh=_MESH,
    compiler_params=pltpu.CompilerParams(needs_layout_passes=False),
    scratch_types=[
        pltpu.VMEM((CHB,), jnp.int32),
        pltpu.VMEM((CHB,), jnp.int32),
        pltpu.VMEM((CHB, PK), jnp.float32),
        pltpu.VMEM((CHB, PK), jnp.float32),
        pltpu.VMEM((CHB * D,), jnp.float32),
        pltpu.VMEM((CHB * D,), jnp.float32),
        pltpu.VMEM((SPW * 2 * D,), jnp.float32),
        pltpu.SemaphoreType.DMA,
        pltpu.SemaphoreType.DMA,
    ],
)
def _title_kernel(title_hbm, tids_hbm, text_hbm, out_hbm,
                  tix0, tix1, trow0, trow1, tex0, tex1, outw, isem, tsem):
    wid = lax.axis_index("s") * NC + lax.axis_index("c")
    cid0 = wid * NCHB

    def stage_idx(cidx, xb, eb):
        pltpu.async_copy(tids_hbm.at[pl.ds(cidx * CHB, CHB)], xb, isem)
        pltpu.async_copy(
            text_hbm.at[pl.ds(cidx * (CHB * D), CHB * D)], eb, isem)

    def wait_idx(cidx, xb, eb):
        pltpu.make_async_copy(
            tids_hbm.at[pl.ds(cidx * CHB, CHB)], xb, isem).wait()
        pltpu.make_async_copy(
            text_hbm.at[pl.ds(cidx * (CHB * D), CHB * D)], eb, isem).wait()

    def fire_gather(xb, tb):
        pltpu.async_copy(title_hbm.at[xb], tb, tsem)

    def wait_gather(xb, tb):
        pltpu.make_async_copy(title_hbm.at[xb], tb, tsem).wait()

    pltpu.sync_copy(tids_hbm.at[pl.ds(cid0 * CHB, CHB)], tix0)
    pltpu.sync_copy(text_hbm.at[pl.ds(cid0 * (CHB * D), CHB * D)], tex0)
    fire_gather(tix0, trow0)
    stage_idx(cid0 + 1, tix1, tex1)

    bufs = ((tix0, trow0, tex0), (tix1, trow1, tex1))

    def body(cc, _):
        for p in range(2):
            c = cc * 2 + p
            cidx = cid0 + c
            xb, tb, eb = bufs[p]
            xbn, tbn, ebn = bufs[1 - p]

            @pl.when(c + 1 < NCHB)
            def _():
                wait_idx(cidx + 1, xbn, ebn)
                fire_gather(xbn, tbn)

            wait_gather(xb, tb)

            for i2 in range(CHB):
                ob = pl.multiple_of(
                    c * (CHB * 2 * D) + i2 * 2 * D, 2 * D)
                outw[pl.ds(ob, 16)] = tb[i2, pl.ds(0, 16)]
                outw[pl.ds(ob + 16, 16)] = tb[i2, pl.ds(16, 16)]
                outw[pl.ds(ob + 32, 16)] = eb[pl.ds(i2 * D, 16)]
                outw[pl.ds(ob + 48, 16)] = eb[pl.ds(i2 * D + 16, 16)]

            @pl.when(c + 2 < NCHB)
            def _():
                stage_idx(cidx + 2, xb, eb)
        return 0

    lax.fori_loop(0, NCHB // 2, body, 0)
    pltpu.sync_copy(outw, out_hbm.at[pl.ds(wid * (SPW * 2 * D), SPW * 2 * D)])


def kernel(title_table, text_table, title_ids, token_ids):
    text_z = text_table.at[0].set(0.0)
    title_p = jnp.pad(title_table, ((0, 7), (0, PK - D)))
    text_flat = _text_kernel(text_z, token_ids)
    flat = _title_kernel(title_p, title_ids, text_flat)
    return flat.reshape(B, 2 * D)
